# speedup vs baseline: 8.1236x; 8.1236x over previous
import jax
import jax.numpy as jnp
import numpy as np
from jax.experimental import pallas as pl

NF = 26
D = 64
NPAIR = NF * (NF - 1) // 2
BLK = 512


def _tc_body(x_ref, o_ref):
    x3 = x_ref[...].reshape(NF, D, BLK)
    off = 0
    for i in range(NF - 1):
        nj = NF - 1 - i
        prod = x3[i + 1:] * x3[i][None]
        o_ref[off:off + nj, :] = jnp.sum(prod, axis=1)
        off += nj


def kernel(x):
    b = x.shape[0]
    xt = x.reshape(b, NF * D).T
    out_t = pl.pallas_call(
        _tc_body,
        grid=(b // BLK,),
        in_specs=[pl.BlockSpec((NF * D, BLK), lambda i: (0, i))],
        out_specs=pl.BlockSpec((NPAIR, BLK), lambda i: (0, i)),
        out_shape=jax.ShapeDtypeStruct((NPAIR, b), jnp.float32),
    )(xt)
    return out_t.T


# speedup vs baseline: 9.7738x; 1.2031x over previous
import jax
import jax.numpy as jnp
import numpy as np
from jax.experimental import pallas as pl

NF = 26
D = 64
NPAIR = NF * (NF - 1) // 2
BLK = 512


def _tc_body(x_ref, o_ref):
    x3 = x_ref[...].reshape(NF, D, BLK)
    off = 0
    for i in range(NF - 1):
        nj = NF - 1 - i
        q = x3[i + 1:]
        p = x3[i]
        acc = q[:, 0:8, :] * p[None, 0:8, :]
        for dv in range(1, D // 8):
            sl = slice(dv * 8, dv * 8 + 8)
            acc = acc + q[:, sl, :] * p[None, sl, :]
        o_ref[off:off + nj, :] = jnp.sum(acc, axis=1)
        off += nj


def kernel(x):
    b = x.shape[0]
    xt = x.reshape(b, NF * D).T.astype(jnp.bfloat16)
    out_t = pl.pallas_call(
        _tc_body,
        grid=(b // BLK,),
        in_specs=[pl.BlockSpec((NF * D, BLK), lambda i: (0, i))],
        out_specs=pl.BlockSpec((NPAIR, BLK), lambda i: (0, i)),
        out_shape=jax.ShapeDtypeStruct((NPAIR, b), jnp.bfloat16),
    )(xt)
    return out_t.T.astype(jnp.float32)
